# baseline (speedup 1.0000x reference)
import functools

import jax
import jax.numpy as jnp
from jax import lax
from jax.experimental import pallas as pl
from jax.experimental.pallas import tpu as pltpu
from jax.experimental.pallas import tpu_sc as plsc

LANES = 16
NUM_CORES = 2
NUM_SUBCORES = 16
NUM_WORKERS = NUM_CORES * NUM_SUBCORES
POS_CHUNK = 32


def _make_emb(batch, seq, vocab, d):
    pos_per_w = seq // NUM_WORKERS
    n_chunks = pos_per_w // POS_CHUNK
    slices_per_row = d // LANES

    mesh = plsc.VectorSubcoreMesh(core_axis_name="c", subcore_axis_name="s")

    @functools.partial(
        pl.kernel,
        mesh=mesh,
        out_type=jax.ShapeDtypeStruct((batch * seq, d), jnp.float32),
        scratch_types=[
            pltpu.VMEM((POS_CHUNK,), jnp.int32),
            pltpu.VMEM((POS_CHUNK, d), jnp.float32),
            pltpu.VMEM((POS_CHUNK, d), jnp.float32),
            pltpu.SemaphoreType.DMA,
        ],
    )
    def emb(x_hbm, tok_hbm, pos_hbm, out_hbm, idx_v, pos_v, rows_v, sem):
        wid = lax.axis_index("s") * NUM_CORES + lax.axis_index("c")
        p0 = wid * pos_per_w
        for chunk in range(n_chunks):
            pbase = p0 + chunk * POS_CHUNK
            pltpu.sync_copy(pos_hbm.at[pl.ds(pbase, POS_CHUNK)], pos_v)
            for b in range(batch):
                row0 = b * seq + pbase
                pltpu.sync_copy(x_hbm.at[pl.ds(row0, POS_CHUNK)], idx_v)
                pltpu.async_copy(tok_hbm.at[idx_v], rows_v, sem).wait()

                def add_row(k, carry):
                    i = k // slices_per_row
                    j = (k % slices_per_row) * LANES
                    plsc.addupdate(
                        rows_v.at[i, pl.ds(j, LANES)],
                        pos_v[i, pl.ds(j, LANES)],
                    )
                    return carry

                lax.fori_loop(0, POS_CHUNK * slices_per_row, add_row, 0)
                pltpu.sync_copy(rows_v, out_hbm.at[pl.ds(row0, POS_CHUNK)])

    return emb


def kernel(x, token_table, pos_table):
    batch, seq = x.shape
    vocab, d = token_table.shape
    xf = x.reshape(batch * seq).astype(jnp.int32)
    emb = _make_emb(batch, seq, vocab, d)
    out = emb(xf, token_table, pos_table)
    return out.reshape(batch, seq, d)


# speedup vs baseline: 1.6328x; 1.6328x over previous
import functools

import jax
import jax.numpy as jnp
from jax import lax
from jax.experimental import pallas as pl
from jax.experimental.pallas import tpu as pltpu
from jax.experimental.pallas import tpu_sc as plsc

LANES = 16
NUM_CORES = 2
NUM_SUBCORES = 16
NUM_WORKERS = NUM_CORES * NUM_SUBCORES
ROW_CHUNK = 16
NBUF = 2


def _make_emb(batch, seq, vocab, d):
    pos_per_w = seq // NUM_WORKERS
    chunks = pos_per_w // ROW_CHUNK
    rounds = batch * chunks
    slices_per_row = d // LANES

    mesh = plsc.VectorSubcoreMesh(core_axis_name="c", subcore_axis_name="s")

    @functools.partial(
        pl.kernel,
        mesh=mesh,
        out_type=jax.ShapeDtypeStruct((batch * seq, d), jnp.float32),
        scratch_types=[
            pltpu.VMEM((batch * pos_per_w,), jnp.int32),
            pltpu.VMEM((pos_per_w, d), jnp.float32),
        ]
        + [pltpu.VMEM((ROW_CHUNK, d), jnp.float32) for _ in range(NBUF)]
        + [pltpu.SemaphoreType.DMA for _ in range(2 * NBUF + 1)],
    )
    def emb(x_hbm, tok_hbm, pos_hbm, out_hbm, idx_v, pos_v, *bufs_sems):
        rows = bufs_sems[:NBUF]
        gsem = bufs_sems[NBUF:2 * NBUF]
        wsem = bufs_sems[2 * NBUF:3 * NBUF]
        psem = bufs_sems[3 * NBUF]

        wid = lax.axis_index("s") * NUM_CORES + lax.axis_index("c")
        p0 = wid * pos_per_w

        pos_cp = pltpu.async_copy(pos_hbm.at[pl.ds(p0, pos_per_w)], pos_v, psem)
        for b in range(batch):
            pltpu.sync_copy(
                x_hbm.at[pl.ds(b * seq + p0, pos_per_w)],
                idx_v.at[pl.ds(b * pos_per_w, pos_per_w)],
            )

        def start_gather(r):
            b, c = divmod(r, chunks)
            idx = idx_v.at[pl.ds(b * pos_per_w + c * ROW_CHUNK, ROW_CHUNK)]
            return pltpu.async_copy(tok_hbm.at[idx], rows[r % NBUF], gsem[r % NBUF])

        gcp = [None] * rounds
        wcp = [None] * rounds
        gcp[0] = start_gather(0)
        pos_cp.wait()
        for r in range(rounds):
            buf = r % NBUF
            if r + 1 < rounds:
                if r + 1 >= NBUF:
                    wcp[r + 1 - NBUF].wait()
                gcp[r + 1] = start_gather(r + 1)
            gcp[r].wait()
            b, c = divmod(r, chunks)

            def add_row(i, carry, _buf=buf, _c=c):
                for j in range(slices_per_row):
                    plsc.addupdate(
                        rows[_buf].at[i, pl.ds(j * LANES, LANES)],
                        pos_v[_c * ROW_CHUNK + i, pl.ds(j * LANES, LANES)],
                    )
                return carry

            lax.fori_loop(0, ROW_CHUNK, add_row, 0)
            wcp[r] = pltpu.async_copy(
                rows[buf],
                out_hbm.at[pl.ds(b * seq + p0 + c * ROW_CHUNK, ROW_CHUNK)],
                wsem[buf],
            )
        for r in range(rounds - NBUF, rounds):
            wcp[r].wait()

    return emb


def kernel(x, token_table, pos_table):
    batch, seq = x.shape
    vocab, d = token_table.shape
    xf = x.reshape(batch * seq).astype(jnp.int32)
    emb = _make_emb(batch, seq, vocab, d)
    out = emb(xf, token_table, pos_table)
    return out.reshape(batch, seq, d)


# speedup vs baseline: 2.1478x; 1.3154x over previous
import functools

import jax
import jax.numpy as jnp
from jax import lax
from jax.experimental import pallas as pl
from jax.experimental.pallas import tpu as pltpu
from jax.experimental.pallas import tpu_sc as plsc

LANES = 16
NUM_CORES = 2
NUM_SUBCORES = 16
NUM_WORKERS = NUM_CORES * NUM_SUBCORES
ROW_CHUNK = 16
NBUF = 5


def _make_emb(batch, seq, vocab, d):
    pos_per_w = seq // NUM_WORKERS
    chunks = pos_per_w // ROW_CHUNK
    rounds = chunks * batch
    slices_per_row = d // LANES

    mesh = plsc.VectorSubcoreMesh(core_axis_name="c", subcore_axis_name="s")

    @functools.partial(
        pl.kernel,
        mesh=mesh,
        out_type=jax.ShapeDtypeStruct((batch * seq, d), jnp.float32),
        scratch_types=[
            pltpu.VMEM((batch * pos_per_w,), jnp.int32),
            pltpu.VMEM((ROW_CHUNK, d), jnp.float32),
            pltpu.VMEM((ROW_CHUNK, d), jnp.float32),
        ]
        + [pltpu.VMEM((ROW_CHUNK, d), jnp.float32) for _ in range(NBUF)]
        + [pltpu.SemaphoreType.DMA for _ in range(2 * NBUF + 2)],
    )
    def emb(x_hbm, tok_hbm, pos_hbm, out_hbm, idx_v, *refs):
        pbuf = refs[:2]
        rows = refs[2:2 + NBUF]
        gsem = refs[2 + NBUF:2 + 2 * NBUF]
        wsem = refs[2 + 2 * NBUF:2 + 3 * NBUF]
        psem = refs[2 + 3 * NBUF:2 + 3 * NBUF + 2]

        wid = lax.axis_index("s") * NUM_CORES + lax.axis_index("c")
        p0 = wid * pos_per_w

        for b in range(batch):
            pltpu.sync_copy(
                x_hbm.at[pl.ds(b * seq + p0, pos_per_w)],
                idx_v.at[pl.ds(b * pos_per_w, pos_per_w)],
            )

        def start_pos(c):
            return pltpu.async_copy(
                pos_hbm.at[pl.ds(p0 + c * ROW_CHUNK, ROW_CHUNK)],
                pbuf[c % 2], psem[c % 2],
            )

        def start_gather(r):
            c, b = divmod(r, batch)
            idx = idx_v.at[pl.ds(b * pos_per_w + c * ROW_CHUNK, ROW_CHUNK)]
            return pltpu.async_copy(tok_hbm.at[idx], rows[r % NBUF], gsem[r % NBUF])

        ahead = NBUF - 2
        pos_cp = [None, None]
        pos_cp[0] = start_pos(0)
        gcp = [None] * rounds
        wcp = [None] * rounds
        for r in range(ahead):
            gcp[r] = start_gather(r)

        for r in range(rounds):
            c, b = divmod(r, batch)
            if r + ahead < rounds:
                if r + ahead - NBUF >= 0:
                    wcp[r + ahead - NBUF].wait()
                gcp[r + ahead] = start_gather(r + ahead)
            if b == 0:
                pos_cp[c % 2].wait()
                if c + 1 < chunks:
                    pos_cp[(c + 1) % 2] = start_pos(c + 1)
            gcp[r].wait()

            def add_row(i, carry, _buf=r % NBUF, _p=c % 2):
                for j in range(slices_per_row):
                    plsc.addupdate(
                        rows[_buf].at[i, pl.ds(j * LANES, LANES)],
                        pbuf[_p][i, pl.ds(j * LANES, LANES)],
                    )
                return carry

            lax.fori_loop(0, ROW_CHUNK, add_row, 0)
            wcp[r] = pltpu.async_copy(
                rows[r % NBUF],
                out_hbm.at[pl.ds(b * seq + p0 + c * ROW_CHUNK, ROW_CHUNK)],
                wsem[r % NBUF],
            )
        for r in range(rounds - NBUF, rounds):
            wcp[r].wait()

    return emb


def kernel(x, token_table, pos_table):
    batch, seq = x.shape
    vocab, d = token_table.shape
    xf = x.reshape(batch * seq).astype(jnp.int32)
    emb = _make_emb(batch, seq, vocab, d)
    out = emb(xf, token_table, pos_table)
    return out.reshape(batch, seq, d)


# speedup vs baseline: 2.1816x; 1.0158x over previous
import functools

import jax
import jax.numpy as jnp
from jax import lax
from jax.experimental import pallas as pl
from jax.experimental.pallas import tpu as pltpu
from jax.experimental.pallas import tpu_sc as plsc

LANES = 16
NUM_CORES = 2
NUM_SUBCORES = 16
NUM_WORKERS = NUM_CORES * NUM_SUBCORES
ROW_CHUNK = 16
NBUF = 5


def _make_emb(batch, seq, vocab, d):
    pos_per_w = seq // NUM_WORKERS
    chunks = pos_per_w // ROW_CHUNK
    rounds = chunks * batch
    slices_per_row = d // LANES

    mesh = plsc.VectorSubcoreMesh(core_axis_name="c", subcore_axis_name="s")

    @functools.partial(
        pl.kernel,
        mesh=mesh,
        out_type=jax.ShapeDtypeStruct((batch * seq, d), jnp.float32),
        scratch_types=[
            pltpu.VMEM((batch * pos_per_w,), jnp.int32),
            pltpu.VMEM((ROW_CHUNK, d), jnp.float32),
            pltpu.VMEM((ROW_CHUNK, d), jnp.float32),
        ]
        + [pltpu.VMEM((ROW_CHUNK, d), jnp.float32) for _ in range(NBUF)]
        + [pltpu.SemaphoreType.DMA for _ in range(2 * NBUF + 3)],
    )
    def emb(x_hbm, tok_hbm, pos_hbm, out_hbm, idx_v, *refs):
        pbuf = refs[:2]
        rows = refs[2:2 + NBUF]
        gsem = refs[2 + NBUF:2 + 2 * NBUF]
        wsem = refs[2 + 2 * NBUF:2 + 3 * NBUF]
        psem = refs[2 + 3 * NBUF:2 + 3 * NBUF + 2]
        isem = refs[2 + 3 * NBUF + 2]

        wid = lax.axis_index("s") * NUM_CORES + lax.axis_index("c")
        p0 = wid * pos_per_w

        idx_cps = [
            pltpu.async_copy(
                x_hbm.at[pl.ds(b * seq + p0, pos_per_w)],
                idx_v.at[pl.ds(b * pos_per_w, pos_per_w)],
                isem,
            )
            for b in range(batch)
        ]

        def start_pos(c):
            return pltpu.async_copy(
                pos_hbm.at[pl.ds(p0 + c * ROW_CHUNK, ROW_CHUNK)],
                pbuf[c % 2], psem[c % 2],
            )

        def start_gather(r):
            c, b = divmod(r, batch)
            idx = idx_v.at[pl.ds(b * pos_per_w + c * ROW_CHUNK, ROW_CHUNK)]
            return pltpu.async_copy(tok_hbm.at[idx], rows[r % NBUF], gsem[r % NBUF])

        ahead = NBUF - 2
        pos_cp = [None, None]
        pos_cp[0] = start_pos(0)
        for cp in idx_cps:
            cp.wait()
        gcp = [None] * rounds
        wcp = [None] * rounds
        for r in range(ahead):
            gcp[r] = start_gather(r)

        for r in range(rounds):
            c, b = divmod(r, batch)
            if r + ahead < rounds:
                if r + ahead - NBUF >= 0:
                    wcp[r + ahead - NBUF].wait()
                gcp[r + ahead] = start_gather(r + ahead)
            if b == 0:
                pos_cp[c % 2].wait()
                if c + 1 < chunks:
                    pos_cp[(c + 1) % 2] = start_pos(c + 1)
            gcp[r].wait()

            def add_row(i, carry, _buf=r % NBUF, _p=c % 2):
                for j in range(slices_per_row):
                    plsc.addupdate(
                        rows[_buf].at[i, pl.ds(j * LANES, LANES)],
                        pbuf[_p][i, pl.ds(j * LANES, LANES)],
                    )
                return carry

            lax.fori_loop(0, ROW_CHUNK, add_row, 0)
            wcp[r] = pltpu.async_copy(
                rows[r % NBUF],
                out_hbm.at[pl.ds(b * seq + p0 + c * ROW_CHUNK, ROW_CHUNK)],
                wsem[r % NBUF],
            )
        for r in range(rounds - NBUF, rounds):
            wcp[r].wait()

    return emb


def kernel(x, token_table, pos_table):
    batch, seq = x.shape
    vocab, d = token_table.shape
    xf = x.reshape(batch * seq).astype(jnp.int32)
    emb = _make_emb(batch, seq, vocab, d)
    out = emb(xf, token_table, pos_table)
    return out.reshape(batch, seq, d)
